# baseline (speedup 1.0000x reference)
import functools

import jax
import jax.numpy as jnp
from jax import lax
from jax.experimental import pallas as pl
from jax.experimental.pallas import tpu as pltpu
from jax.experimental.pallas import tpu_sc as plsc

N = 100000
D = 128
H = 128
NB = 64
BLK = 10000
BLKE = 5000
NW = 32
CH_G = 13
CH_ROWS = CH_G * 16

PART_ROWS = (40000, 60000)
PART_OFF = (0, 40000)


def _part_consts(np_rows):
    g_p = np_rows // 16
    gw_lo = g_p // NW
    n_hi = g_p - gw_lo * NW
    gw_hi = gw_lo + 1
    n_full = gw_lo // CH_G
    return dict(G_P=g_p, GW_LO=gw_lo, N_HI=n_hi, GW_HI=gw_hi,
                N_FULL=n_full, WSLICE=gw_lo * 16, BBUF=(gw_hi + 1) * 16)


def _gate_body(h_ref, w1_ref, b1_ref, w2t_ref, b2_ref, w_ref, m_ref, msc):
    i = pl.program_id(0)
    act = jnp.dot(h_ref[...], w1_ref[...],
                  preferred_element_type=jnp.float32) + b1_ref[...]
    act = act * jax.nn.sigmoid(act)
    w = jnp.sum(act * w2t_ref[...], axis=1, keepdims=True) + b2_ref[0, 0]
    w_ref[...] = w
    bm = jnp.max(w)
    prev = jnp.where(i == 0, -jnp.inf, msc[0, 0])
    msc[0, 0] = jnp.maximum(prev, bm)

    @pl.when(i == pl.num_programs(0) - 1)
    def _():
        m_ref[...] = jnp.full((1, 16), msc[0, 0], dtype=jnp.float32)


def _e16_body(w_ref, m_ref, e_ref):
    e = jnp.exp(w_ref[...] - m_ref[0, 0])
    e_ref[...] = jnp.broadcast_to(e, (BLKE, 16))


def _pool_sc_body(row_off, C, h_hbm, e_hbm, b_hbm, num_hbm, den_hbm,
                  hbuf, ebuf, bbuf, acc, dacc, sem0, sem1):
    GW_LO = C["GW_LO"]
    GW_HI = C["GW_HI"]
    N_HI = C["N_HI"]
    N_FULL = C["N_FULL"]
    WSLICE = C["WSLICE"]
    cid = lax.axis_index("c")
    sid = lax.axis_index("s")
    wid = sid * 2 + cid
    hi = wid < N_HI
    base_g = jnp.where(hi, wid * GW_HI, N_HI * GW_HI + (wid - N_HI) * GW_LO)
    ng = jnp.where(hi, GW_HI, GW_LO)
    base_row = base_g * 16
    rem = ng - N_FULL * CH_G

    pltpu.sync_copy(b_hbm.at[pl.ds(row_off + base_row, WSLICE)],
                    bbuf.at[pl.ds(0, WSLICE)])

    @pl.when(hi)
    def _():
        pltpu.sync_copy(b_hbm.at[pl.ds(row_off + base_row + WSLICE, 16)],
                        bbuf.at[pl.ds(WSLICE, 16)])

    z16 = jnp.zeros((16,), jnp.float32)

    def zbody(r, carry):
        for j in range(D // 16):
            acc[r, pl.ds(j * 16, 16)] = z16
        dacc[r, pl.ds(0, 16)] = z16
        return carry
    lax.fori_loop(0, NB, zbody, 0)

    hsem = sem0
    esem = sem1

    def start_rows(row, slot):
        pltpu.async_copy(h_hbm.at[pl.ds(row_off + row, CH_ROWS)],
                         hbuf.at[slot], hsem)
        pltpu.async_copy(e_hbm.at[pl.ds(row, CH_ROWS)], ebuf.at[slot], esem)

    def wait_chunk(slot):
        pltpu.make_async_copy(h_hbm.at[pl.ds(0, CH_ROWS)],
                              hbuf.at[slot], hsem).wait()
        pltpu.make_async_copy(e_hbm.at[pl.ds(0, CH_ROWS)],
                              ebuf.at[slot], esem).wait()

    def process(slot, cbase_g, g_lo, g_hi):
        def gbody(g, carry):
            bg = bbuf[pl.ds((cbase_g + g) * 16, 16)]
            b0 = bg[0]
            b15 = bg[15]
            row0 = g * 16

            @pl.when(b0 == b15)
            def _():
                evs = [ebuf[slot, row0 + r, pl.ds(0, 16)] for r in range(16)]
                for j in range(D // 16):
                    s = hbuf[slot, row0, pl.ds(j * 16, 16)] * evs[0]
                    for r in range(1, 16):
                        s = s + hbuf[slot, row0 + r, pl.ds(j * 16, 16)] * evs[r]
                    plsc.addupdate(acc.at[b0, pl.ds(j * 16, 16)], s)
                vsum = evs[0]
                for r in range(1, 16):
                    vsum = vsum + evs[r]
                plsc.addupdate(dacc.at[b0, pl.ds(0, 16)], vsum)

            @pl.when(b0 != b15)
            def _():
                def rbody(r16, carry2):
                    bwin = bbuf[pl.ds((cbase_g + g) * 16 + r16, 16)]
                    b_r = bwin[0]
                    ev = ebuf[slot, row0 + r16, pl.ds(0, 16)]
                    plsc.addupdate(dacc.at[b_r, pl.ds(0, 16)], ev)
                    for j in range(D // 16):
                        v = hbuf[slot, row0 + r16, pl.ds(j * 16, 16)] * ev
                        plsc.addupdate(acc.at[b_r, pl.ds(j * 16, 16)], v)
                    return carry2
                lax.fori_loop(0, 16, rbody, 0)
            return carry
        lax.fori_loop(g_lo, g_hi, gbody, 0)

    start_rows(base_row, 0)

    def cbody(c, carry):
        slot = lax.rem(c, 2)
        wait_chunk(slot)

        @pl.when(c + 1 < N_FULL)
        def _():
            start_rows(base_row + (c + 1) * CH_ROWS, 1 - slot)

        @pl.when(c + 1 == N_FULL)
        def _():
            start_rows(base_row + (ng - CH_G) * 16, 1 - slot)
        process(slot, c * CH_G, 0, CH_G)
        return carry
    lax.fori_loop(0, N_FULL, cbody, 0)

    tslot = lax.rem(N_FULL, 2)
    wait_chunk(tslot)
    process(tslot, ng - CH_G, CH_G - rem, CH_G)

    pltpu.sync_copy(acc, num_hbm.at[wid])
    pltpu.sync_copy(dacc, den_hbm.at[wid])


def _combine_body(n0_ref, n1_ref, d0_ref, d1_ref, m0_ref, m1_ref, out_ref):
    m0 = m0_ref[0, 0]
    m1 = m1_ref[0, 0]
    mg = jnp.maximum(m0, m1)
    s0 = jnp.exp(m0 - mg)
    s1 = jnp.exp(m1 - mg)
    s = s0 * jnp.sum(n0_ref[...], axis=0) + s1 * jnp.sum(n1_ref[...], axis=0)
    d = s0 * jnp.sum(d0_ref[...], axis=0) + s1 * jnp.sum(d1_ref[...], axis=0)
    dcol = jnp.sum(d, axis=1, keepdims=True) * (1.0 / 16.0)
    out_ref[...] = s / (dcol + 1e-6)


def _gate_call(h, W1, b1r, w2t, b2r, part):
    np_rows = PART_ROWS[part]
    blk0 = PART_OFF[part] // BLK
    return pl.pallas_call(
        _gate_body,
        grid=(np_rows // BLK,),
        in_specs=[
            pl.BlockSpec((BLK, D), lambda i, b0=blk0: (i + b0, 0)),
            pl.BlockSpec((D, H), lambda i: (0, 0)),
            pl.BlockSpec((1, H), lambda i: (0, 0)),
            pl.BlockSpec((1, H), lambda i: (0, 0)),
            pl.BlockSpec((1, 1), lambda i: (0, 0)),
        ],
        out_specs=[
            pl.BlockSpec((BLK, 1), lambda i: (i, 0)),
            pl.BlockSpec((1, 16), lambda i: (0, 0)),
        ],
        out_shape=[
            jax.ShapeDtypeStruct((np_rows, 1), jnp.float32),
            jax.ShapeDtypeStruct((1, 16), jnp.float32),
        ],
        scratch_shapes=[pltpu.SMEM((1, 1), jnp.float32)],
    )(h, W1, b1r, w2t, b2r)


def _e16_call(w, m):
    np_rows = w.shape[0]
    return pl.pallas_call(
        _e16_body,
        grid=(np_rows // BLKE,),
        in_specs=[
            pl.BlockSpec((BLKE, 1), lambda i: (i, 0)),
            pl.BlockSpec((1, 16), lambda i: (0, 0)),
        ],
        out_specs=pl.BlockSpec((BLKE, 16), lambda i: (i, 0)),
        out_shape=jax.ShapeDtypeStruct((np_rows, 16), jnp.float32),
    )(w, m)


def _pool_call(h, e16, bi32, part):
    C = _part_consts(PART_ROWS[part])
    pool = pl.kernel(
        functools.partial(_pool_sc_body, PART_OFF[part], C),
        out_type=[
            jax.ShapeDtypeStruct((NW, NB, D), jnp.float32),
            jax.ShapeDtypeStruct((NW, NB, 16), jnp.float32),
        ],
        mesh=plsc.VectorSubcoreMesh(core_axis_name="c", subcore_axis_name="s"),
        scratch_types=[
            pltpu.VMEM((2, CH_ROWS, D), jnp.float32),
            pltpu.VMEM((2, CH_ROWS, 16), jnp.float32),
            pltpu.VMEM((C["BBUF"],), jnp.int32),
            pltpu.VMEM((NB, D), jnp.float32),
            pltpu.VMEM((NB, 16), jnp.float32),
            pltpu.SemaphoreType.DMA,
            pltpu.SemaphoreType.DMA,
        ],
    )
    return pool(h, e16, bi32)


@jax.jit
def kernel(h, batch, W1, b1, W2, b2):
    b1r = b1.reshape(1, H)
    w2t = W2.reshape(1, H)
    b2r = b2.reshape(1, 1)
    bi32 = batch.astype(jnp.int32)

    w0, m0 = _gate_call(h, W1, b1r, w2t, b2r, 0)
    e0 = _e16_call(w0, m0)
    num0, den0 = _pool_call(h, e0, bi32, 0)

    w1, m1 = _gate_call(h, W1, b1r, w2t, b2r, 1)
    e1 = _e16_call(w1, m1)
    num1, den1 = _pool_call(h, e1, bi32, 1)

    out = pl.pallas_call(
        _combine_body,
        in_specs=[
            pl.BlockSpec((NW, NB, D), lambda: (0, 0, 0)),
            pl.BlockSpec((NW, NB, D), lambda: (0, 0, 0)),
            pl.BlockSpec((NW, NB, 16), lambda: (0, 0, 0)),
            pl.BlockSpec((NW, NB, 16), lambda: (0, 0, 0)),
            pl.BlockSpec((1, 16), lambda: (0, 0)),
            pl.BlockSpec((1, 16), lambda: (0, 0)),
        ],
        out_specs=pl.BlockSpec((NB, D), lambda: (0, 0)),
        out_shape=jax.ShapeDtypeStruct((NB, D), jnp.float32),
    )(num0, num1, den0, den1, m0, m1)
    return out


# speedup vs baseline: 1.0043x; 1.0043x over previous
import functools

import jax
import jax.numpy as jnp
from jax import lax
from jax.experimental import pallas as pl
from jax.experimental.pallas import tpu as pltpu
from jax.experimental.pallas import tpu_sc as plsc

N = 100000
D = 128
H = 128
NB = 64
BLK = 10000
BLKE = 5000
NW = 32
CH_G = 13
CH_ROWS = CH_G * 16

PART_ROWS = (60000, 40000)
PART_OFF = (0, 60000)


def _part_consts(np_rows):
    g_p = np_rows // 16
    gw_lo = g_p // NW
    n_hi = g_p - gw_lo * NW
    gw_hi = gw_lo + 1
    n_full = gw_lo // CH_G
    return dict(G_P=g_p, GW_LO=gw_lo, N_HI=n_hi, GW_HI=gw_hi,
                N_FULL=n_full, WSLICE=gw_lo * 16, BBUF=(gw_hi + 1) * 16)


def _gate_body(h_ref, w1_ref, b1_ref, w2t_ref, b2_ref, w_ref, m_ref, msc):
    i = pl.program_id(0)
    act = jnp.dot(h_ref[...], w1_ref[...],
                  preferred_element_type=jnp.float32) + b1_ref[...]
    act = act * jax.nn.sigmoid(act)
    w = jnp.sum(act * w2t_ref[...], axis=1, keepdims=True) + b2_ref[0, 0]
    w_ref[...] = w
    bm = jnp.max(w)
    prev = jnp.where(i == 0, -jnp.inf, msc[0, 0])
    msc[0, 0] = jnp.maximum(prev, bm)

    @pl.when(i == pl.num_programs(0) - 1)
    def _():
        m_ref[...] = jnp.full((1, 16), msc[0, 0], dtype=jnp.float32)


def _e16_body(w_ref, m_ref, e_ref):
    e = jnp.exp(w_ref[...] - m_ref[0, 0])
    e_ref[...] = jnp.broadcast_to(e, (BLKE, 16))


def _pool_sc_body(row_off, C, h_hbm, e_hbm, b_hbm, num_hbm, den_hbm,
                  hbuf, ebuf, bbuf, acc, dacc, sem0, sem1):
    GW_LO = C["GW_LO"]
    GW_HI = C["GW_HI"]
    N_HI = C["N_HI"]
    N_FULL = C["N_FULL"]
    WSLICE = C["WSLICE"]
    cid = lax.axis_index("c")
    sid = lax.axis_index("s")
    wid = sid * 2 + cid
    hi = wid < N_HI
    base_g = jnp.where(hi, wid * GW_HI, N_HI * GW_HI + (wid - N_HI) * GW_LO)
    ng = jnp.where(hi, GW_HI, GW_LO)
    base_row = base_g * 16
    rem = ng - N_FULL * CH_G

    pltpu.sync_copy(b_hbm.at[pl.ds(row_off + base_row, WSLICE)],
                    bbuf.at[pl.ds(0, WSLICE)])

    @pl.when(hi)
    def _():
        pltpu.sync_copy(b_hbm.at[pl.ds(row_off + base_row + WSLICE, 16)],
                        bbuf.at[pl.ds(WSLICE, 16)])

    z16 = jnp.zeros((16,), jnp.float32)

    def zbody(r, carry):
        for j in range(D // 16):
            acc[r, pl.ds(j * 16, 16)] = z16
        dacc[r, pl.ds(0, 16)] = z16
        return carry
    lax.fori_loop(0, NB, zbody, 0)

    hsem = sem0
    esem = sem1

    def start_rows(row, slot):
        pltpu.async_copy(h_hbm.at[pl.ds(row_off + row, CH_ROWS)],
                         hbuf.at[slot], hsem)
        pltpu.async_copy(e_hbm.at[pl.ds(row, CH_ROWS)], ebuf.at[slot], esem)

    def wait_chunk(slot):
        pltpu.make_async_copy(h_hbm.at[pl.ds(0, CH_ROWS)],
                              hbuf.at[slot], hsem).wait()
        pltpu.make_async_copy(e_hbm.at[pl.ds(0, CH_ROWS)],
                              ebuf.at[slot], esem).wait()

    def process(slot, cbase_g, g_lo, g_hi):
        def gbody(g, carry):
            bg = bbuf[pl.ds((cbase_g + g) * 16, 16)]
            b0 = bg[0]
            b15 = bg[15]
            row0 = g * 16

            @pl.when(b0 == b15)
            def _():
                evs = [ebuf[slot, row0 + r, pl.ds(0, 16)] for r in range(16)]
                for j in range(D // 16):
                    s = hbuf[slot, row0, pl.ds(j * 16, 16)] * evs[0]
                    for r in range(1, 16):
                        s = s + hbuf[slot, row0 + r, pl.ds(j * 16, 16)] * evs[r]
                    plsc.addupdate(acc.at[b0, pl.ds(j * 16, 16)], s)
                vsum = evs[0]
                for r in range(1, 16):
                    vsum = vsum + evs[r]
                plsc.addupdate(dacc.at[b0, pl.ds(0, 16)], vsum)

            @pl.when(b0 != b15)
            def _():
                def rbody(r16, carry2):
                    bwin = bbuf[pl.ds((cbase_g + g) * 16 + r16, 16)]
                    b_r = bwin[0]
                    ev = ebuf[slot, row0 + r16, pl.ds(0, 16)]
                    plsc.addupdate(dacc.at[b_r, pl.ds(0, 16)], ev)
                    for j in range(D // 16):
                        v = hbuf[slot, row0 + r16, pl.ds(j * 16, 16)] * ev
                        plsc.addupdate(acc.at[b_r, pl.ds(j * 16, 16)], v)
                    return carry2
                lax.fori_loop(0, 16, rbody, 0)
            return carry
        lax.fori_loop(g_lo, g_hi, gbody, 0)

    start_rows(base_row, 0)

    def cbody(c, carry):
        slot = lax.rem(c, 2)
        wait_chunk(slot)

        @pl.when(c + 1 < N_FULL)
        def _():
            start_rows(base_row + (c + 1) * CH_ROWS, 1 - slot)

        @pl.when(c + 1 == N_FULL)
        def _():
            start_rows(base_row + (ng - CH_G) * 16, 1 - slot)
        process(slot, c * CH_G, 0, CH_G)
        return carry
    lax.fori_loop(0, N_FULL, cbody, 0)

    tslot = lax.rem(N_FULL, 2)
    wait_chunk(tslot)
    process(tslot, ng - CH_G, CH_G - rem, CH_G)

    pltpu.sync_copy(acc, num_hbm.at[wid])
    pltpu.sync_copy(dacc, den_hbm.at[wid])


def _combine_body(n0_ref, n1_ref, d0_ref, d1_ref, m0_ref, m1_ref, out_ref):
    m0 = m0_ref[0, 0]
    m1 = m1_ref[0, 0]
    mg = jnp.maximum(m0, m1)
    s0 = jnp.exp(m0 - mg)
    s1 = jnp.exp(m1 - mg)
    s = s0 * jnp.sum(n0_ref[...], axis=0) + s1 * jnp.sum(n1_ref[...], axis=0)
    d = s0 * jnp.sum(d0_ref[...], axis=0) + s1 * jnp.sum(d1_ref[...], axis=0)
    dcol = jnp.sum(d, axis=1, keepdims=True) * (1.0 / 16.0)
    out_ref[...] = s / (dcol + 1e-6)


def _gate_call(h, W1, b1r, w2t, b2r, part):
    np_rows = PART_ROWS[part]
    blk0 = PART_OFF[part] // BLK
    return pl.pallas_call(
        _gate_body,
        grid=(np_rows // BLK,),
        in_specs=[
            pl.BlockSpec((BLK, D), lambda i, b0=blk0: (i + b0, 0)),
            pl.BlockSpec((D, H), lambda i: (0, 0)),
            pl.BlockSpec((1, H), lambda i: (0, 0)),
            pl.BlockSpec((1, H), lambda i: (0, 0)),
            pl.BlockSpec((1, 1), lambda i: (0, 0)),
        ],
        out_specs=[
            pl.BlockSpec((BLK, 1), lambda i: (i, 0)),
            pl.BlockSpec((1, 16), lambda i: (0, 0)),
        ],
        out_shape=[
            jax.ShapeDtypeStruct((np_rows, 1), jnp.float32),
            jax.ShapeDtypeStruct((1, 16), jnp.float32),
        ],
        scratch_shapes=[pltpu.SMEM((1, 1), jnp.float32)],
    )(h, W1, b1r, w2t, b2r)


def _e16_call(w, m):
    np_rows = w.shape[0]
    return pl.pallas_call(
        _e16_body,
        grid=(np_rows // BLKE,),
        in_specs=[
            pl.BlockSpec((BLKE, 1), lambda i: (i, 0)),
            pl.BlockSpec((1, 16), lambda i: (0, 0)),
        ],
        out_specs=pl.BlockSpec((BLKE, 16), lambda i: (i, 0)),
        out_shape=jax.ShapeDtypeStruct((np_rows, 16), jnp.float32),
    )(w, m)


def _pool_call(h, e16, bi32, part):
    C = _part_consts(PART_ROWS[part])
    pool = pl.kernel(
        functools.partial(_pool_sc_body, PART_OFF[part], C),
        out_type=[
            jax.ShapeDtypeStruct((NW, NB, D), jnp.float32),
            jax.ShapeDtypeStruct((NW, NB, 16), jnp.float32),
        ],
        mesh=plsc.VectorSubcoreMesh(core_axis_name="c", subcore_axis_name="s"),
        scratch_types=[
            pltpu.VMEM((2, CH_ROWS, D), jnp.float32),
            pltpu.VMEM((2, CH_ROWS, 16), jnp.float32),
            pltpu.VMEM((C["BBUF"],), jnp.int32),
            pltpu.VMEM((NB, D), jnp.float32),
            pltpu.VMEM((NB, 16), jnp.float32),
            pltpu.SemaphoreType.DMA,
            pltpu.SemaphoreType.DMA,
        ],
    )
    return pool(h, e16, bi32)


@jax.jit
def kernel(h, batch, W1, b1, W2, b2):
    b1r = b1.reshape(1, H)
    w2t = W2.reshape(1, H)
    b2r = b2.reshape(1, 1)
    bi32 = batch.astype(jnp.int32)

    w0, m0 = _gate_call(h, W1, b1r, w2t, b2r, 0)
    e0 = _e16_call(w0, m0)
    num0, den0 = _pool_call(h, e0, bi32, 0)

    w1, m1 = _gate_call(h, W1, b1r, w2t, b2r, 1)
    e1 = _e16_call(w1, m1)
    num1, den1 = _pool_call(h, e1, bi32, 1)

    out = pl.pallas_call(
        _combine_body,
        in_specs=[
            pl.BlockSpec((NW, NB, D), lambda: (0, 0, 0)),
            pl.BlockSpec((NW, NB, D), lambda: (0, 0, 0)),
            pl.BlockSpec((NW, NB, 16), lambda: (0, 0, 0)),
            pl.BlockSpec((NW, NB, 16), lambda: (0, 0, 0)),
            pl.BlockSpec((1, 16), lambda: (0, 0)),
            pl.BlockSpec((1, 16), lambda: (0, 0)),
        ],
        out_specs=pl.BlockSpec((NB, D), lambda: (0, 0)),
        out_shape=jax.ShapeDtypeStruct((NB, D), jnp.float32),
    )(num0, num1, den0, den1, m0, m1)
    return out


# speedup vs baseline: 1.0418x; 1.0373x over previous
import functools

import jax
import jax.numpy as jnp
from jax import lax
from jax.experimental import pallas as pl
from jax.experimental.pallas import tpu as pltpu
from jax.experimental.pallas import tpu_sc as plsc

N = 100000
D = 128
H = 128
NB = 64
NPART = 2
N_P = N // NPART
BLK = 10000
GRID_P = N_P // BLK
BLKE = 5000
GRIDE_P = N_P // BLKE

NW = 32
G_P = N_P // 16
GW_LO = G_P // NW
N_HI = G_P - GW_LO * NW
GW_HI = GW_LO + 1
CH_G = 13
CH_ROWS = CH_G * 16
N_FULL = GW_LO // CH_G
WSLICE = GW_LO * 16
BBUF = (GW_HI + 1) * 16


def _gate_body(h_ref, w1_ref, b1_ref, w2t_ref, b2_ref, w_ref, m_ref, msc):
    i = pl.program_id(0)
    act = jnp.dot(h_ref[...], w1_ref[...],
                  preferred_element_type=jnp.float32) + b1_ref[...]
    act = act * jax.nn.sigmoid(act)
    w = jnp.sum(act * w2t_ref[...], axis=1, keepdims=True) + b2_ref[0, 0]
    w_ref[...] = w
    bm = jnp.max(w)
    prev = jnp.where(i == 0, -jnp.inf, msc[0, 0])
    msc[0, 0] = jnp.maximum(prev, bm)

    @pl.when(i == GRID_P - 1)
    def _():
        m_ref[...] = jnp.full((1, 16), msc[0, 0], dtype=jnp.float32)


def _e16_body(w_ref, m_ref, e_ref):
    e = jnp.exp(w_ref[...] - m_ref[0, 0])
    e_ref[...] = jnp.broadcast_to(e, (BLKE, 16))


def _pool_sc_body(row_off, h_hbm, e_hbm, b_hbm, num_hbm, den_hbm,
                  hbuf, ebuf, bbuf, acc, dacc, sem0, sem1):
    cid = lax.axis_index("c")
    sid = lax.axis_index("s")
    wid = sid * 2 + cid
    hi = wid < N_HI
    base_g = jnp.where(hi, wid * GW_HI, N_HI * GW_HI + (wid - N_HI) * GW_LO)
    ng = jnp.where(hi, GW_HI, GW_LO)
    base_row = base_g * 16
    rem = ng - N_FULL * CH_G

    pltpu.sync_copy(b_hbm.at[pl.ds(row_off + base_row, WSLICE)],
                    bbuf.at[pl.ds(0, WSLICE)])

    @pl.when(hi)
    def _():
        pltpu.sync_copy(b_hbm.at[pl.ds(row_off + base_row + WSLICE, 16)],
                        bbuf.at[pl.ds(WSLICE, 16)])

    z16 = jnp.zeros((16,), jnp.float32)

    def zbody(r, carry):
        for j in range(D // 16):
            acc[r, pl.ds(j * 16, 16)] = z16
        dacc[r, pl.ds(0, 16)] = z16
        return carry
    lax.fori_loop(0, NB, zbody, 0)

    hsem = sem0
    esem = sem1

    def start_rows(row, slot):
        pltpu.async_copy(h_hbm.at[pl.ds(row_off + row, CH_ROWS)],
                         hbuf.at[slot], hsem)
        pltpu.async_copy(e_hbm.at[pl.ds(row, CH_ROWS)], ebuf.at[slot], esem)

    def wait_chunk(slot):
        pltpu.make_async_copy(h_hbm.at[pl.ds(0, CH_ROWS)],
                              hbuf.at[slot], hsem).wait()
        pltpu.make_async_copy(e_hbm.at[pl.ds(0, CH_ROWS)],
                              ebuf.at[slot], esem).wait()

    def process(slot, cbase_g, g_lo, g_hi):
        def gbody(g, carry):
            bg = bbuf[pl.ds((cbase_g + g) * 16, 16)]
            b0 = bg[0]
            b15 = bg[15]
            row0 = g * 16

            @pl.when(b0 == b15)
            def _():
                evs = [ebuf[slot, row0 + r, pl.ds(0, 16)] for r in range(16)]
                for j in range(D // 16):
                    s = hbuf[slot, row0, pl.ds(j * 16, 16)] * evs[0]
                    for r in range(1, 16):
                        s = s + hbuf[slot, row0 + r, pl.ds(j * 16, 16)] * evs[r]
                    plsc.addupdate(acc.at[b0, pl.ds(j * 16, 16)], s)
                vsum = evs[0]
                for r in range(1, 16):
                    vsum = vsum + evs[r]
                plsc.addupdate(dacc.at[b0, pl.ds(0, 16)], vsum)

            @pl.when(b0 != b15)
            def _():
                def rbody(r16, carry2):
                    bwin = bbuf[pl.ds((cbase_g + g) * 16 + r16, 16)]
                    b_r = bwin[0]
                    ev = ebuf[slot, row0 + r16, pl.ds(0, 16)]
                    plsc.addupdate(dacc.at[b_r, pl.ds(0, 16)], ev)
                    for j in range(D // 16):
                        v = hbuf[slot, row0 + r16, pl.ds(j * 16, 16)] * ev
                        plsc.addupdate(acc.at[b_r, pl.ds(j * 16, 16)], v)
                    return carry2
                lax.fori_loop(0, 16, rbody, 0)
            return carry
        lax.fori_loop(g_lo, g_hi, gbody, 0)

    start_rows(base_row, 0)

    def cbody(c, carry):
        slot = lax.rem(c, 2)
        wait_chunk(slot)

        @pl.when(c + 1 < N_FULL)
        def _():
            start_rows(base_row + (c + 1) * CH_ROWS, 1 - slot)

        @pl.when(c + 1 == N_FULL)
        def _():
            start_rows(base_row + (ng - CH_G) * 16, 1 - slot)
        process(slot, c * CH_G, 0, CH_G)
        return carry
    lax.fori_loop(0, N_FULL, cbody, 0)

    tslot = lax.rem(N_FULL, 2)
    wait_chunk(tslot)
    process(tslot, ng - CH_G, CH_G - rem, CH_G)

    pltpu.sync_copy(acc, num_hbm.at[wid])
    pltpu.sync_copy(dacc, den_hbm.at[wid])


def _combine_body(n0_ref, n1_ref, d0_ref, d1_ref, m0_ref, m1_ref, out_ref):
    m0 = m0_ref[0, 0]
    m1 = m1_ref[0, 0]
    mg = jnp.maximum(m0, m1)
    s0 = jnp.exp(m0 - mg)
    s1 = jnp.exp(m1 - mg)
    s = s0 * jnp.sum(n0_ref[...], axis=0) + s1 * jnp.sum(n1_ref[...], axis=0)
    d = s0 * jnp.sum(d0_ref[...], axis=0) + s1 * jnp.sum(d1_ref[...], axis=0)
    dcol = jnp.sum(d, axis=1, keepdims=True) * (1.0 / 16.0)
    out_ref[...] = s / (dcol + 1e-6)


def _gate_call(h, W1, b1r, w2t, b2r, part):
    return pl.pallas_call(
        _gate_body,
        grid=(GRID_P,),
        in_specs=[
            pl.BlockSpec((BLK, D), lambda i, p=part: (i + p * GRID_P, 0)),
            pl.BlockSpec((D, H), lambda i: (0, 0)),
            pl.BlockSpec((1, H), lambda i: (0, 0)),
            pl.BlockSpec((1, H), lambda i: (0, 0)),
            pl.BlockSpec((1, 1), lambda i: (0, 0)),
        ],
        out_specs=[
            pl.BlockSpec((BLK, 1), lambda i: (i, 0)),
            pl.BlockSpec((1, 16), lambda i: (0, 0)),
        ],
        out_shape=[
            jax.ShapeDtypeStruct((N_P, 1), jnp.float32),
            jax.ShapeDtypeStruct((1, 16), jnp.float32),
        ],
        scratch_shapes=[pltpu.SMEM((1, 1), jnp.float32)],
    )(h, W1, b1r, w2t, b2r)


def _e16_call(w, m):
    return pl.pallas_call(
        _e16_body,
        grid=(GRIDE_P,),
        in_specs=[
            pl.BlockSpec((BLKE, 1), lambda i: (i, 0)),
            pl.BlockSpec((1, 16), lambda i: (0, 0)),
        ],
        out_specs=pl.BlockSpec((BLKE, 16), lambda i: (i, 0)),
        out_shape=jax.ShapeDtypeStruct((N_P, 16), jnp.float32),
    )(w, m)


def _pool_call(h, e16, bi32, part):
    pool = pl.kernel(
        functools.partial(_pool_sc_body, part * N_P),
        out_type=[
            jax.ShapeDtypeStruct((NW, NB, D), jnp.float32),
            jax.ShapeDtypeStruct((NW, NB, 16), jnp.float32),
        ],
        mesh=plsc.VectorSubcoreMesh(core_axis_name="c", subcore_axis_name="s"),
        scratch_types=[
            pltpu.VMEM((2, CH_ROWS, D), jnp.float32),
            pltpu.VMEM((2, CH_ROWS, 16), jnp.float32),
            pltpu.VMEM((BBUF,), jnp.int32),
            pltpu.VMEM((NB, D), jnp.float32),
            pltpu.VMEM((NB, 16), jnp.float32),
            pltpu.SemaphoreType.DMA,
            pltpu.SemaphoreType.DMA,
        ],
    )
    return pool(h, e16, bi32)


@jax.jit
def kernel(h, batch, W1, b1, W2, b2):
    b1r = b1.reshape(1, H)
    w2t = W2.reshape(1, H)
    b2r = b2.reshape(1, 1)
    bi32 = batch.astype(jnp.int32)

    w0, m0 = _gate_call(h, W1, b1r, w2t, b2r, 0)
    e0 = _e16_call(w0, m0)
    num0, den0 = _pool_call(h, e0, bi32, 0)

    w1, m1 = _gate_call(h, W1, b1r, w2t, b2r, 1)
    e1 = _e16_call(w1, m1)
    num1, den1 = _pool_call(h, e1, bi32, 1)

    out = pl.pallas_call(
        _combine_body,
        in_specs=[
            pl.BlockSpec((NW, NB, D), lambda: (0, 0, 0)),
            pl.BlockSpec((NW, NB, D), lambda: (0, 0, 0)),
            pl.BlockSpec((NW, NB, 16), lambda: (0, 0, 0)),
            pl.BlockSpec((NW, NB, 16), lambda: (0, 0, 0)),
            pl.BlockSpec((1, 16), lambda: (0, 0)),
            pl.BlockSpec((1, 16), lambda: (0, 0)),
        ],
        out_specs=pl.BlockSpec((NB, D), lambda: (0, 0)),
        out_shape=jax.ShapeDtypeStruct((NB, D), jnp.float32),
    )(num0, num1, den0, den1, m0, m1)
    return out
